# baseline (speedup 1.0000x reference)
import functools
import math

import jax
import jax.numpy as jnp
from jax import lax
from jax.experimental import pallas as pl
from jax.experimental.pallas import tpu as pltpu
from jax.experimental.pallas import tpu_sc as plsc

_VOCAB = 100000
_EMBED = 64
_BATCH = 16384
_LMIN = math.log(0.1)
_LMAX = math.log(10.0)
_CLIP = math.sqrt(2.0)

_NC = 2
_NS = 16
_NW = _NC * _NS
_ROWS_PER_W = _BATCH // _NW
_CHUNK = 128
_NCHUNK = _ROWS_PER_W // _CHUNK
_L = 16


def _tile_body(wi_hbm, wj_hbm, wn_hbm, mu_h, mup_h, mun_h, ls_h, lsp_h,
               lsn_h, out_h, idx_i, idx_j, idx_n,
               b00, b01, b02, b03, b04, b05,
               b10, b11, b12, b13, b14, b15, ovec,
               sem0, sem1):
    bufs0 = (b00, b01, b02, b03, b04, b05)
    bufs1 = (b10, b11, b12, b13, b14, b15)
    wid = lax.axis_index("s") * _NC + lax.axis_index("c")

    pltpu.sync_copy(wi_hbm.at[wid], idx_i)
    pltpu.sync_copy(wj_hbm.at[wid], idx_j)
    pltpu.sync_copy(wn_hbm.at[wid], idx_n)

    tables = (mu_h, mup_h, mun_h, ls_h, lsp_h, lsn_h)
    idxs = (idx_i, idx_j, idx_n, idx_i, idx_j, idx_n)
    slots = ((bufs0, sem0), (bufs1, sem1))

    def fire(c, slot):
        bufs, sem = slots[slot]
        return [pltpu.async_copy(t.at[ix.at[c]], b, sem)
                for t, ix, b in zip(tables, idxs, bufs)]

    lane = lax.iota(jnp.int32, 16)
    _gdn = lax.GatherDimensionNumbers(
        offset_dims=(), collapsed_slice_dims=(0,), start_index_map=(0,))

    def _shuffle(v, idx):
        return lax.gather(v, idx[:, None], _gdn, slice_sizes=(1,),
                          mode=lax.GatherScatterMode.PROMISE_IN_BOUNDS)

    def chunk_sum(bufs, total):
        bmi, bmj, bmn, bsi, bsj, bsn = bufs

        def row_body(r, vtot):
            acc = jnp.zeros((_L,), jnp.float32)
            for kk in range(_EMBED // _L):
                sl = pl.ds(kk * _L, _L)
                mi = jnp.clip(bmi[r, sl], -_CLIP, _CLIP)
                mj = jnp.clip(bmj[r, sl], -_CLIP, _CLIP)
                mn = jnp.clip(bmn[r, sl], -_CLIP, _CLIP)
                li = jnp.clip(bsi[r, sl], _LMIN, _LMAX)
                lj = jnp.clip(bsj[r, sl], _LMIN, _LMAX)
                ln = jnp.clip(bsn[r, sl], _LMIN, _LMAX)
                rinv = jnp.exp(-li)
                dj = mi - mj
                dn = mi - mn
                acc = acc + (jnp.exp(lj) - jnp.exp(ln) + dj * dj
                             - dn * dn) * rinv + (lj - ln)
            for sh in (8, 4, 2, 1):
                acc = acc + _shuffle(acc, lane ^ sh)
            return vtot + jnp.maximum(1.0 + 0.5 * acc, 0.0)

        return lax.fori_loop(0, _CHUNK, row_body, total, unroll=8)

    total = jnp.zeros((_L,), jnp.float32)
    pending = fire(0, 0)
    for c in range(_NCHUNK):
        nxt = fire(c + 1, (c + 1) % 2) if c + 1 < _NCHUNK else None
        for d in pending:
            d.wait()
        total = chunk_sum(slots[c % 2][0], total)
        pending = nxt

    ovec[...] = total
    pltpu.sync_copy(ovec, out_h.at[wid])


@jax.jit
def _sc_loss_partials(wi, wj, wn, mu, mu_pos, mu_neg, ls, ls_pos, ls_neg):
    mesh = plsc.VectorSubcoreMesh(core_axis_name="c", subcore_axis_name="s")
    f32 = jnp.float32
    grid_kernel = pl.kernel(
        _tile_body,
        mesh=mesh,
        out_type=jax.ShapeDtypeStruct((_NW, _L), f32),
        scratch_types=[
            pltpu.VMEM((_NCHUNK, _CHUNK), jnp.int32),
            pltpu.VMEM((_NCHUNK, _CHUNK), jnp.int32),
            pltpu.VMEM((_NCHUNK, _CHUNK), jnp.int32),
            *[pltpu.VMEM((_CHUNK, _EMBED), f32) for _ in range(12)],
            pltpu.VMEM((_L,), f32),
            pltpu.SemaphoreType.DMA,
            pltpu.SemaphoreType.DMA,
        ],
        compiler_params=pltpu.CompilerParams(use_tc_tiling_on_sc=False),
    )
    return grid_kernel(wi, wj, wn, mu, mu_pos, mu_neg, ls, ls_pos, ls_neg)


def kernel(words_i, words_j, words_n, mu, mu_pos, mu_neg, log_sigma,
           log_sigma_pos, log_sigma_neg):
    wi = words_i.astype(jnp.int32).reshape(_NW, _NCHUNK, _CHUNK)
    wj = words_j.astype(jnp.int32).reshape(_NW, _NCHUNK, _CHUNK)
    wn = words_n.astype(jnp.int32).reshape(_NW, _NCHUNK, _CHUNK)
    partials = _sc_loss_partials(wi, wj, wn, mu, mu_pos, mu_neg, log_sigma,
                                 log_sigma_pos, log_sigma_neg)
    return jnp.sum(partials) * (1.0 / (_BATCH * _L))


# speedup vs baseline: 1.1693x; 1.1693x over previous
import math

import jax
import jax.numpy as jnp
from jax import lax
from jax.experimental import pallas as pl
from jax.experimental.pallas import tpu as pltpu
from jax.experimental.pallas import tpu_sc as plsc

_VOCAB = 100000
_EMBED = 64
_BATCH = 16384
_LMIN = math.log(0.1)
_LMAX = math.log(10.0)
_CLIP = math.sqrt(2.0)

_NC = 2
_NS = 16
_NW = _NC * _NS
_ROWS_PER_W = _BATCH // _NW
_CHUNK = 128
_NCHUNK = _ROWS_PER_W // _CHUNK
_L = 16
_D2 = 2 * _EMBED


def _tile_body(wi_hbm, wj_hbm, wn_hbm, ti_h, tj_h, tn_h, out_h,
               idx_i, idx_j, idx_n,
               b00, b01, b02, b10, b11, b12, ovec, sem0, sem1):
    bufs0 = (b00, b01, b02)
    bufs1 = (b10, b11, b12)
    wid = lax.axis_index("s") * _NC + lax.axis_index("c")

    pltpu.sync_copy(wi_hbm.at[wid], idx_i)
    pltpu.sync_copy(wj_hbm.at[wid], idx_j)
    pltpu.sync_copy(wn_hbm.at[wid], idx_n)

    tables = (ti_h, tj_h, tn_h)
    idxs = (idx_i, idx_j, idx_n)
    slots = ((bufs0, sem0), (bufs1, sem1))

    def fire(c, slot):
        bufs, sem = slots[slot]
        return [pltpu.async_copy(t.at[ix.at[c]], b, sem)
                for t, ix, b in zip(tables, idxs, bufs)]

    lane = lax.iota(jnp.int32, 16)
    _gdn = lax.GatherDimensionNumbers(
        offset_dims=(), collapsed_slice_dims=(0,), start_index_map=(0,))

    def _shuffle(v, idx):
        return lax.gather(v, idx[:, None], _gdn, slice_sizes=(1,),
                          mode=lax.GatherScatterMode.PROMISE_IN_BOUNDS)

    def chunk_sum(bufs, total):
        bi, bj, bn = bufs

        def row_body(r, vtot):
            acc = jnp.zeros((_L,), jnp.float32)
            for kk in range(_EMBED // _L):
                msl = pl.ds(kk * _L, _L)
                ssl = pl.ds(_EMBED + kk * _L, _L)
                mi = jnp.clip(bi[r, msl], -_CLIP, _CLIP)
                mj = jnp.clip(bj[r, msl], -_CLIP, _CLIP)
                mn = jnp.clip(bn[r, msl], -_CLIP, _CLIP)
                li = jnp.clip(bi[r, ssl], _LMIN, _LMAX)
                lj = jnp.clip(bj[r, ssl], _LMIN, _LMAX)
                ln = jnp.clip(bn[r, ssl], _LMIN, _LMAX)
                rinv = jnp.exp(-li)
                dj = mi - mj
                dn = mi - mn
                acc = acc + (jnp.exp(lj) - jnp.exp(ln) + dj * dj
                             - dn * dn) * rinv + (lj - ln)
            for sh in (8, 4, 2, 1):
                acc = acc + _shuffle(acc, lane ^ sh)
            return vtot + jnp.maximum(1.0 + 0.5 * acc, 0.0)

        return lax.fori_loop(0, _CHUNK, row_body, total, unroll=8)

    total = jnp.zeros((_L,), jnp.float32)
    pending = fire(0, 0)
    for c in range(_NCHUNK):
        nxt = fire(c + 1, (c + 1) % 2) if c + 1 < _NCHUNK else None
        for d in pending:
            d.wait()
        total = chunk_sum(slots[c % 2][0], total)
        pending = nxt

    ovec[...] = total
    pltpu.sync_copy(ovec, out_h.at[wid])


@jax.jit
def _sc_loss_partials(wi, wj, wn, tab_i, tab_j, tab_n):
    mesh = plsc.VectorSubcoreMesh(core_axis_name="c", subcore_axis_name="s")
    f32 = jnp.float32
    grid_kernel = pl.kernel(
        _tile_body,
        mesh=mesh,
        out_type=jax.ShapeDtypeStruct((_NW, _L), f32),
        scratch_types=[
            pltpu.VMEM((_NCHUNK, _CHUNK), jnp.int32),
            pltpu.VMEM((_NCHUNK, _CHUNK), jnp.int32),
            pltpu.VMEM((_NCHUNK, _CHUNK), jnp.int32),
            *[pltpu.VMEM((_CHUNK, _D2), f32) for _ in range(6)],
            pltpu.VMEM((_L,), f32),
            pltpu.SemaphoreType.DMA,
            pltpu.SemaphoreType.DMA,
        ],
        compiler_params=pltpu.CompilerParams(use_tc_tiling_on_sc=True),
    )
    return grid_kernel(wi, wj, wn, tab_i, tab_j, tab_n)


def kernel(words_i, words_j, words_n, mu, mu_pos, mu_neg, log_sigma,
           log_sigma_pos, log_sigma_neg):
    wi = words_i.astype(jnp.int32).reshape(_NW, _NCHUNK, _CHUNK)
    wj = words_j.astype(jnp.int32).reshape(_NW, _NCHUNK, _CHUNK)
    wn = words_n.astype(jnp.int32).reshape(_NW, _NCHUNK, _CHUNK)
    tab_i = jnp.concatenate([mu, log_sigma], axis=1)
    tab_j = jnp.concatenate([mu_pos, log_sigma_pos], axis=1)
    tab_n = jnp.concatenate([mu_neg, log_sigma_neg], axis=1)
    partials = _sc_loss_partials(wi, wj, wn, tab_i, tab_j, tab_n)
    return jnp.sum(partials) * (1.0 / (_BATCH * _L))
